# baseline (speedup 1.0000x reference)
import functools

import jax
import jax.numpy as jnp
from jax import lax
from jax.experimental import pallas as pl
from jax.experimental.pallas import tpu as pltpu
from jax.experimental.pallas import tpu_sc as plsc

_VOCAB = 50257
_EMBED = 1024
_MAX_SEQ = 2048
_BATCH = 4
_NC = 2
_NS = 16
_NW = _NC * _NS
_SEQ_PER_W = _MAX_SEQ // _NW
_CHUNK = 16
_NQ = _SEQ_PER_W // _CHUNK
_NCHUNK = _NQ * _BATCH
_NBUF = 5
_AHEAD = 3
_LANES = 16


def _emb_body(ids_hbm, tok_hbm, pos_hbm, out_hbm,
              idx_v, tok0, tok1, tok2, tok3, tok4, pos0, pos1, gsem, wsem, psem):
    wid = lax.axis_index("s") * _NC + lax.axis_index("c")
    seq0 = wid * _SEQ_PER_W

    tokbufs = (tok0, tok1, tok2, tok3, tok4)
    posbufs = (pos0, pos1)

    idx_cps = [pltpu.async_copy(ids_hbm.at[b, wid], idx_v.at[b], gsem)
               for b in range(_BATCH)]
    for cp in idx_cps:
        cp.wait()

    def fire_gather(c):
        q, b = divmod(c, _BATCH)
        return pltpu.async_copy(
            tok_hbm.at[idx_v.at[b, q]], tokbufs[c % _NBUF], gsem)

    def fire_pos(q):
        return pltpu.async_copy(
            pos_hbm.at[pl.ds(seq0 + q * _CHUNK, _CHUNK)], posbufs[q % 2], psem)

    def add_rows(tokbuf, posbuf):
        def row_body(r, _):
            for j in range(_EMBED // _LANES):
                s = pl.ds(j * _LANES, _LANES)
                tokbuf[r, s] = tokbuf[r, s] + posbuf[r, s]
            return 0
        lax.fori_loop(0, _CHUNK, row_body, 0)

    pos_cps = [fire_pos(0), None]
    gathers = [None] * _NBUF
    writes = [None] * _NBUF
    for k in range(_AHEAD):
        gathers[k] = fire_gather(k)

    for c in range(_NCHUNK):
        q, b = divmod(c, _BATCH)
        buf = c % _NBUF

        if c + _AHEAD < _NCHUNK:
            nbuf = (c + _AHEAD) % _NBUF
            if writes[nbuf] is not None:
                writes[nbuf].wait()
                writes[nbuf] = None
            gathers[nbuf] = fire_gather(c + _AHEAD)

        if b == 0:
            pos_cps[q % 2].wait()
            if q + 1 < _NQ:
                pos_cps[(q + 1) % 2] = fire_pos(q + 1)

        gathers[buf].wait()
        add_rows(tokbufs[buf], posbufs[q % 2])

        writes[buf] = pltpu.async_copy(
            tokbufs[buf], out_hbm.at[b, pl.ds(seq0 + q * _CHUNK, _CHUNK)], wsem)

    for w in writes:
        if w is not None:
            w.wait()


@jax.jit
def _embed(ids, tok_table, pos_table):
    mesh = plsc.VectorSubcoreMesh(core_axis_name="c", subcore_axis_name="s")
    run = functools.partial(
        pl.kernel,
        out_type=jax.ShapeDtypeStruct((_BATCH, _MAX_SEQ, _EMBED), jnp.float32),
        mesh=mesh,
        scratch_types=[
            pltpu.VMEM((_BATCH, _NQ, _CHUNK), jnp.int32),
            pltpu.VMEM((_CHUNK, _EMBED), jnp.float32),
            pltpu.VMEM((_CHUNK, _EMBED), jnp.float32),
            pltpu.VMEM((_CHUNK, _EMBED), jnp.float32),
            pltpu.VMEM((_CHUNK, _EMBED), jnp.float32),
            pltpu.VMEM((_CHUNK, _EMBED), jnp.float32),
            pltpu.VMEM((_CHUNK, _EMBED), jnp.float32),
            pltpu.VMEM((_CHUNK, _EMBED), jnp.float32),
            pltpu.SemaphoreType.DMA,
            pltpu.SemaphoreType.DMA,
            pltpu.SemaphoreType.DMA,
        ],
    )(_emb_body)
    return run(ids, tok_table, pos_table)


def kernel(input_ids, token_table, pos_table):
    ids = input_ids.astype(jnp.int32).reshape(_BATCH, _NW, _NQ, _CHUNK)
    return _embed(ids, token_table, pos_table)


# speedup vs baseline: 1.0015x; 1.0015x over previous
import functools

import jax
import jax.numpy as jnp
from jax import lax
from jax.experimental import pallas as pl
from jax.experimental.pallas import tpu as pltpu
from jax.experimental.pallas import tpu_sc as plsc

_VOCAB = 50257
_EMBED = 1024
_MAX_SEQ = 2048
_BATCH = 4
_NC = 2
_NS = 16
_NW = _NC * _NS
_SEQ_PER_W = _MAX_SEQ // _NW
_CHUNK = 16
_NQ = _SEQ_PER_W // _CHUNK
_NCHUNK = _NQ * _BATCH
_NBUF = 5
_AHEAD = 3
_LANES = 16


def _emb_body(ids_hbm, tok_hbm, pos_hbm, out_hbm,
              idx_v, tok0, tok1, tok2, tok3, tok4, pos0, pos1, gsem, wsem, psem):
    wid = lax.axis_index("s") * _NC + lax.axis_index("c")
    seq0 = wid * _SEQ_PER_W

    tokbufs = (tok0, tok1, tok2, tok3, tok4)
    posbufs = (pos0, pos1)

    idx_cps = [pltpu.async_copy(ids_hbm.at[b, pl.ds(seq0, _SEQ_PER_W)],
                                idx_v.at[b], gsem)
               for b in range(_BATCH)]
    for cp in idx_cps:
        cp.wait()

    def fire_gather(c):
        q, b = divmod(c, _BATCH)
        return pltpu.async_copy(
            tok_hbm.at[idx_v.at[b, pl.ds(q * _CHUNK, _CHUNK)]],
            tokbufs[c % _NBUF], gsem)

    def fire_pos(q):
        return pltpu.async_copy(
            pos_hbm.at[pl.ds(seq0 + q * _CHUNK, _CHUNK)], posbufs[q % 2], psem)

    def add_rows(tokbuf, posbuf):
        def row_body(r, _):
            for j in range(_EMBED // _LANES):
                s = pl.ds(j * _LANES, _LANES)
                tokbuf[r, s] = tokbuf[r, s] + posbuf[r, s]
            return 0
        lax.fori_loop(0, _CHUNK, row_body, 0)

    pos_cps = [fire_pos(0), None]
    gathers = [None] * _NBUF
    writes = [None] * _NBUF
    for k in range(_AHEAD):
        gathers[k] = fire_gather(k)

    for c in range(_NCHUNK):
        q, b = divmod(c, _BATCH)
        buf = c % _NBUF

        if c + _AHEAD < _NCHUNK:
            nbuf = (c + _AHEAD) % _NBUF
            if writes[nbuf] is not None:
                writes[nbuf].wait()
                writes[nbuf] = None
            gathers[nbuf] = fire_gather(c + _AHEAD)

        if b == 0:
            pos_cps[q % 2].wait()
            if q + 1 < _NQ:
                pos_cps[(q + 1) % 2] = fire_pos(q + 1)

        gathers[buf].wait()
        add_rows(tokbufs[buf], posbufs[q % 2])

        writes[buf] = pltpu.async_copy(
            tokbufs[buf], out_hbm.at[b, pl.ds(seq0 + q * _CHUNK, _CHUNK)], wsem)

    for w in writes:
        if w is not None:
            w.wait()


@jax.jit
def _embed(ids, tok_table, pos_table):
    mesh = plsc.VectorSubcoreMesh(core_axis_name="c", subcore_axis_name="s")
    run = functools.partial(
        pl.kernel,
        out_type=jax.ShapeDtypeStruct((_BATCH, _MAX_SEQ, _EMBED), jnp.float32),
        mesh=mesh,
        scratch_types=[
            pltpu.VMEM((_BATCH, _SEQ_PER_W), jnp.int32),
            pltpu.VMEM((_CHUNK, _EMBED), jnp.float32),
            pltpu.VMEM((_CHUNK, _EMBED), jnp.float32),
            pltpu.VMEM((_CHUNK, _EMBED), jnp.float32),
            pltpu.VMEM((_CHUNK, _EMBED), jnp.float32),
            pltpu.VMEM((_CHUNK, _EMBED), jnp.float32),
            pltpu.VMEM((_CHUNK, _EMBED), jnp.float32),
            pltpu.VMEM((_CHUNK, _EMBED), jnp.float32),
            pltpu.SemaphoreType.DMA,
            pltpu.SemaphoreType.DMA,
            pltpu.SemaphoreType.DMA,
        ],
    )(_emb_body)
    return run(ids, tok_table, pos_table)


def kernel(input_ids, token_table, pos_table):
    return _embed(input_ids.astype(jnp.int32), token_table, pos_table)


# speedup vs baseline: 1.1029x; 1.1012x over previous
import functools

import jax
import jax.numpy as jnp
from jax import lax
from jax.experimental import pallas as pl
from jax.experimental.pallas import tpu as pltpu
from jax.experimental.pallas import tpu_sc as plsc

_VOCAB = 50257
_EMBED = 1024
_MAX_SEQ = 2048
_BATCH = 4
_NC = 2
_NS = 16
_NW = _NC * _NS
_SEQ_PER_W = _MAX_SEQ // _NW
_CHUNK = 16
_NQ = _SEQ_PER_W // _CHUNK
_NCHUNK = _NQ * _BATCH
_NBUF = 5
_AHEAD = 3
_LANES = 16


def _emb_body(ids_hbm, tok_hbm, pos_hbm, out_hbm,
              idx_v, tok0, tok1, tok2, tok3, tok4, pos0, pos1, gsem, wsem, psem):
    wid = lax.axis_index("s") * _NC + lax.axis_index("c")
    seq0 = wid * _SEQ_PER_W

    tokbufs = (tok0, tok1, tok2, tok3, tok4)
    posbufs = (pos0, pos1)

    idx_cps = [pltpu.async_copy(ids_hbm.at[b, pl.ds(seq0, _SEQ_PER_W)],
                                idx_v.at[b], gsem)
               for b in range(_BATCH)]
    for cp in idx_cps:
        cp.wait()

    def fire_gather(c):
        q, b = divmod(c, _BATCH)
        return pltpu.async_copy(
            tok_hbm.at[idx_v.at[b, pl.ds(q * _CHUNK, _CHUNK)]],
            tokbufs[c % _NBUF], gsem)

    def fire_pos(q):
        return pltpu.async_copy(
            pos_hbm.at[pl.ds(seq0 + q * _CHUNK, _CHUNK)], posbufs[q % 2], psem)

    def add_rows(tokbuf, posbuf):
        def row_body(r, _):
            for j in range(_EMBED // _LANES):
                s = pl.ds(j * _LANES, _LANES)
                plsc.addupdate(tokbuf.at[r, s], posbuf[r, s])
            return 0
        lax.fori_loop(0, _CHUNK, row_body, 0)

    pos_cps = [fire_pos(0), None]
    gathers = [None] * _NBUF
    writes = [None] * _NBUF
    for k in range(_AHEAD):
        gathers[k] = fire_gather(k)

    for c in range(_NCHUNK):
        q, b = divmod(c, _BATCH)
        buf = c % _NBUF

        if c + _AHEAD < _NCHUNK:
            nbuf = (c + _AHEAD) % _NBUF
            if writes[nbuf] is not None:
                writes[nbuf].wait()
                writes[nbuf] = None
            gathers[nbuf] = fire_gather(c + _AHEAD)

        if b == 0:
            pos_cps[q % 2].wait()
            if q + 1 < _NQ:
                pos_cps[(q + 1) % 2] = fire_pos(q + 1)

        gathers[buf].wait()
        add_rows(tokbufs[buf], posbufs[q % 2])

        writes[buf] = pltpu.async_copy(
            tokbufs[buf], out_hbm.at[b, pl.ds(seq0 + q * _CHUNK, _CHUNK)], wsem)

    for w in writes:
        if w is not None:
            w.wait()


@jax.jit
def _embed(ids, tok_table, pos_table):
    mesh = plsc.VectorSubcoreMesh(core_axis_name="c", subcore_axis_name="s")
    run = functools.partial(
        pl.kernel,
        out_type=jax.ShapeDtypeStruct((_BATCH, _MAX_SEQ, _EMBED), jnp.float32),
        mesh=mesh,
        scratch_types=[
            pltpu.VMEM((_BATCH, _SEQ_PER_W), jnp.int32),
            pltpu.VMEM((_CHUNK, _EMBED), jnp.float32),
            pltpu.VMEM((_CHUNK, _EMBED), jnp.float32),
            pltpu.VMEM((_CHUNK, _EMBED), jnp.float32),
            pltpu.VMEM((_CHUNK, _EMBED), jnp.float32),
            pltpu.VMEM((_CHUNK, _EMBED), jnp.float32),
            pltpu.VMEM((_CHUNK, _EMBED), jnp.float32),
            pltpu.VMEM((_CHUNK, _EMBED), jnp.float32),
            pltpu.SemaphoreType.DMA,
            pltpu.SemaphoreType.DMA,
            pltpu.SemaphoreType.DMA,
        ],
    )(_emb_body)
    return run(ids, tok_table, pos_table)


def kernel(input_ids, token_table, pos_table):
    return _embed(input_ids.astype(jnp.int32), token_table, pos_table)
